# baseline (speedup 1.0000x reference)
import functools

import jax
import jax.numpy as jnp
from jax import lax
from jax.experimental import pallas as pl
from jax.experimental.pallas import tpu as pltpu
from jax.experimental.pallas import tpu_sc as plsc

Q = 1024
N = 65536
D = 256
QB = 256
KB = 2048

KNN = 30
NW = 32
ROWS_PER_W = Q // NW
CAPL = 64
CAP = 224
CBUF = 16 * CAPL + 16
BIG = 3.0e38


def _dist_body(q_ref, k_ref, out_ref):
    q = q_ref[...]
    k = k_ref[...]
    q2 = jnp.sum(q * q, axis=1, keepdims=True)
    ones = jnp.ones((8, D), jnp.float32)
    k2row = lax.dot_general(ones, k * k, (((1,), (1,)), ((), ())),
                            preferred_element_type=jnp.float32)
    qk = lax.dot_general(q, k, (((1,), (1,)), ((), ())),
                         preferred_element_type=jnp.float32)
    out_ref[...] = (q2 + k2row[0:1, :]) - 2.0 * qk


def _sq_distances(queries, keys):
    grid = (N // KB, Q // QB)
    return pl.pallas_call(
        _dist_body,
        grid=grid,
        in_specs=[
            pl.BlockSpec((QB, D), lambda j, i: (i, 0)),
            pl.BlockSpec((KB, D), lambda j, i: (j, 0)),
        ],
        out_specs=pl.BlockSpec((QB, KB), lambda j, i: (i, j)),
        out_shape=jax.ShapeDtypeStruct((Q, N), jnp.float32),
    )(queries, keys)


def _extract_top30(candbuf, w, iota16):
    nv = (w + 15) // 16
    bigv = jnp.full((16,), BIG)

    def scan_min(i, acc):
        v = candbuf[pl.ds(i * 16, 16)]
        lm = (i * 16 + iota16) < w
        return jnp.minimum(acc, jnp.where(lm, v, bigv))

    def cond(state):
        remaining, _, _, _ = state
        return remaining > 0

    def body(state):
        remaining, o0, o1, _ = state
        m = jnp.min(lax.fori_loop(0, nv, scan_min, bigv))
        mv = jnp.full((16,), m)

        def rm_body(i, c):
            v = candbuf[pl.ds(i * 16, 16)]
            lm = (i * 16 + iota16) < w
            vm = jnp.where(lm, v, bigv)
            eq = vm == mv
            candbuf[pl.ds(i * 16, 16)] = jnp.where(eq, bigv, vm)
            return c + jnp.sum(eq.astype(jnp.int32))

        c = lax.fori_loop(0, nv, rm_body, jnp.int32(0))
        cc = jnp.minimum(c, remaining)
        pos0 = KNN - remaining
        sel0 = (iota16 >= pos0) & (iota16 < pos0 + cc)
        sel1 = (iota16 + 16 >= pos0) & (iota16 + 16 < pos0 + cc)
        o0 = jnp.where(sel0, mv, o0)
        o1 = jnp.where(sel1, mv, o1)
        return remaining - cc, o0, o1, m

    zero = jnp.zeros((16,), jnp.float32)
    state = (jnp.int32(KNN), zero, zero, jnp.float32(0.0))
    _, o0, o1, last_m = lax.while_loop(cond, body, state)
    return o0, o1, last_m


HALF = N // 2
IDXM = 16 * CAPL - 1


def _sc_body(sq_hbm, out_hbm, hbuf, candbuf, buf2, outbuf, s0, s1, s2):
    wid = lax.axis_index("s") * 2 + lax.axis_index("c")
    iota16 = lax.iota(jnp.int32, 16)
    bigv = jnp.full((16,), BIG)
    zi = jnp.zeros((16,), jnp.int32)
    sems = (s0, s1, s2)
    row0 = wid * ROWS_PER_W

    def fire(j, half, slot):
        pltpu.async_copy(
            sq_hbm.at[row0 + j, pl.ds(half * HALF, HALF)],
            hbuf.at[pl.ds(slot * HALF, HALF)], sems[slot])

    def wait(slot):
        pltpu.make_async_copy(
            sq_hbm.at[row0, pl.ds(0, HALF)],
            hbuf.at[pl.ds(slot * HALF, HALF)], sems[slot]).wait()

    def process_row(j, sa, sb, snext, fire_next):
        bufA = hbuf.at[pl.ds(sa * HALF, HALF)]
        bufB = hbuf.at[pl.ds(sb * HALF, HALF)]
        wait(sa)

        @plsc.parallel_loop(0, HALF // 32, unroll=8, carry=(bigv, bigv))
        def p1_accs(i, accs):
            a0, a1 = accs
            a0 = jnp.minimum(a0, bufA[pl.ds(i * 16, 16)])
            a1 = jnp.minimum(a1, bufA[pl.ds(HALF // 2 + i * 16, 16)])
            return a0, a1

        a0, a1 = p1_accs
        m32 = jnp.maximum(jnp.max(a0), jnp.max(a1))
        mv = jnp.full((16,), m32)
        wait(sb)
        if fire_next:
            fire(j + 1, 0, snext)

        @plsc.parallel_loop(0, HALF // 16, unroll=4, carry=zi)
        def p2_cnt(i, cnt):
            va = bufA[pl.ds(i * 16, 16)]
            ma = va <= mv
            idxa = (iota16 + (cnt << 4)) & IDXM
            plsc.store_scatter(candbuf, [idxa], va, mask=ma)
            cnt = cnt + ma.astype(jnp.int32)
            vb = bufB[pl.ds(i * 16, 16)]
            mb = vb <= mv
            idxb = (iota16 + (cnt << 4)) & IDXM
            plsc.store_scatter(candbuf, [idxb], vb, mask=mb)
            return cnt + mb.astype(jnp.int32)

        cnt = p2_cnt
        maxc = jnp.max(cnt)

        def fast_path():
            def cb(jv, w):
                v = candbuf[pl.ds(jv * 16, 16)]
                valid = cnt > jv
                pos = jnp.cumsum(valid.astype(jnp.int32))
                plsc.store_scatter(buf2, [(w - 1) + pos], v, mask=valid)
                return w + jnp.max(pos)

            w = lax.fori_loop(0, maxc, cb, jnp.int32(0))
            return _extract_top30(buf2, w, iota16)

        def slow_path():
            def append4(buf, base, w):
                for t in range(4):
                    v = buf[pl.ds(base + t * 16, 16)]
                    mask = v <= mv
                    pos = jnp.cumsum(mask.astype(jnp.int32))
                    plsc.store_scatter(buf2, [(w - 1) + pos], v, mask=mask)
                    w = w + jnp.max(pos)
                return w

            def compress(w):
                o0, o1, _ = _extract_top30(buf2, w, iota16)
                buf2[pl.ds(0, 16)] = o0
                buf2[pl.ds(16, 16)] = o1
                return jnp.int32(KNN)

            def make_p2e(buf):
                def p2e(g, w):
                    base = g * 64
                    v0 = buf[pl.ds(base, 16)]
                    v1 = buf[pl.ds(base + 16, 16)]
                    v2 = buf[pl.ds(base + 32, 16)]
                    v3 = buf[pl.ds(base + 48, 16)]
                    mn = jnp.minimum(jnp.minimum(v0, v1),
                                     jnp.minimum(v2, v3))
                    hit = jnp.any(mn <= mv)
                    w = lax.cond(hit, lambda w_: append4(buf, base, w_),
                                 lambda w_: w_, w)
                    w = lax.cond(w >= CAP, compress, lambda w_: w_, w)
                    return w

                return p2e

            w = lax.fori_loop(0, HALF // 64, make_p2e(bufA), jnp.int32(0))
            w = lax.fori_loop(0, HALF // 64, make_p2e(bufB), w)
            return _extract_top30(buf2, w, iota16)

        o0, o1, _ = lax.cond(maxc <= CAPL, fast_path, slow_path)
        outbuf[pl.ds(0, 16)] = o0
        outbuf[pl.ds(16, 16)] = o1
        pltpu.sync_copy(outbuf, out_hbm.at[row0 + j])

    fire(0, 0, 0)
    fire(0, 1, 1)

    def rows3(g, carry):
        j = g * 3
        process_row(j, 0, 1, 2, True)
        fire(j + 1, 1, 0)
        process_row(j + 1, 2, 0, 1, True)
        fire(j + 2, 1, 2)
        process_row(j + 2, 1, 2, 0, True)
        fire(j + 3, 1, 1)
        return carry

    lax.fori_loop(0, (ROWS_PER_W - 2) // 3, rows3, jnp.int32(0))
    j = ROWS_PER_W - 2
    process_row(j, 0, 1, 2, True)
    fire(j + 1, 1, 0)
    process_row(j + 1, 2, 0, 1, False)


def _sc_top30(sq):
    mesh = plsc.VectorSubcoreMesh(core_axis_name="c", subcore_axis_name="s")
    f = functools.partial(
        pl.kernel, _sc_body, mesh=mesh,
        compiler_params=pltpu.CompilerParams(needs_layout_passes=False),
        out_type=jax.ShapeDtypeStruct((Q, 32), jnp.float32),
        scratch_types=[
            pltpu.VMEM((3 * HALF,), jnp.float32),
            pltpu.VMEM((16 * CAPL,), jnp.float32),
            pltpu.VMEM((CBUF,), jnp.float32),
            pltpu.VMEM((32,), jnp.float32),
            pltpu.SemaphoreType.DMA,
            pltpu.SemaphoreType.DMA,
            pltpu.SemaphoreType.DMA,
        ],
    )
    return f()(sq)


def _final_body(t30_ref, q_ref, pW1_ref, pb1_ref, pW2_ref, pb2_ref, pW3_ref,
                pb3_ref, tW1_ref, tb1_ref, tW2_ref, tb2_ref, tW3_ref, tb3_ref,
                out_ref):
    eps = 0.001
    L = 5.0
    dm0 = 0.001
    t30 = t30_ref[...]
    kmask = lax.broadcasted_iota(jnp.int32, (Q, 32), 1) < KNN
    d = jnp.sqrt(jnp.maximum(t30, 1e-12))
    d2 = d * d
    dm = 0.99 * dm0 + 0.01 * (
        jnp.sum(jnp.where(kmask, d2, 0.0), axis=1, keepdims=True) / KNN)
    ksum = jnp.sum(jnp.where(kmask, eps / (d2 / dm + eps), 0.0),
                   axis=1, keepdims=True)
    episodic = 1.0 / jnp.sqrt(ksum + eps)

    q = q_ref[...]

    def mlp(W1, b1, W2, b2, W3, b3s):
        h = lax.dot_general(q, W1, (((1,), (1,)), ((), ())),
                            preferred_element_type=jnp.float32) + b1
        h = jnp.maximum(h, 0.0)
        h = lax.dot_general(h, W2, (((1,), (1,)), ((), ())),
                            preferred_element_type=jnp.float32) + b2
        h = jnp.maximum(h, 0.0)
        return jnp.sum(h * W3, axis=1, keepdims=True) + b3s

    t_out = mlp(tW1_ref[...], tb1_ref[...], tW2_ref[...], tb2_ref[...],
                tW3_ref[...], tb3_ref[0, 0])
    p_out = mlp(pW1_ref[...], pb1_ref[...], pW2_ref[...], pb2_ref[...],
                pW3_ref[...], pb3_ref[0, 0])
    pe = (t_out - p_out) ** 2
    mu = jnp.mean(pe)
    var = jnp.mean((pe - mu) ** 2)
    std = jnp.sqrt(var + 1e-8)
    alpha = jnp.clip(pe / std, 1.0, L)
    out_ref[...] = episodic * alpha


def _final(t30, queries, pW1, pb1, pW2, pb2, pW3, pb3,
           tW1, tb1, tW2, tb2, tW3, tb3):
    args = (t30, queries,
            pW1, pb1.reshape(1, -1), pW2, pb2.reshape(1, -1),
            pW3, pb3.reshape(1, -1),
            tW1, tb1.reshape(1, -1), tW2, tb2.reshape(1, -1),
            tW3, tb3.reshape(1, -1))
    out = pl.pallas_call(
        _final_body,
        out_shape=jax.ShapeDtypeStruct((Q, 1), jnp.float32),
    )(*args)
    return out.reshape(Q)


def kernel(queries, keys, pW1, pb1, pW2, pb2, pW3, pb3,
           tW1, tb1, tW2, tb2, tW3, tb3):
    sq = _sq_distances(queries, keys)
    t30 = _sc_top30(sq)
    return _final(t30, queries, pW1, pb1, pW2, pb2, pW3, pb3,
                  tW1, tb1, tW2, tb2, tW3, tb3)


# speedup vs baseline: 1.6024x; 1.6024x over previous
import functools

import jax
import jax.numpy as jnp
from jax import lax
from jax.experimental import pallas as pl
from jax.experimental.pallas import tpu as pltpu
from jax.experimental.pallas import tpu_sc as plsc

Q = 1024
N = 65536
D = 256
QB = 256
KB = 2048

KNN = 30
NW = 32
ROWS_PER_W = Q // NW
CAPL = 64
CAP = 224
CBUF = 16 * CAPL + 16
BIG = 3.0e38


def _dist_body(q_ref, k_ref, out_ref):
    q = q_ref[...]
    k = k_ref[...]
    q2 = jnp.sum(q * q, axis=1, keepdims=True)
    ones = jnp.ones((8, D), jnp.float32)
    k2row = lax.dot_general(ones, k * k, (((1,), (1,)), ((), ())),
                            preferred_element_type=jnp.float32)
    qk = lax.dot_general(q, k, (((1,), (1,)), ((), ())),
                         preferred_element_type=jnp.float32)
    out_ref[...] = (q2 + k2row[0:1, :]) - 2.0 * qk


def _sq_distances(queries, keys):
    grid = (N // KB, Q // QB)
    return pl.pallas_call(
        _dist_body,
        grid=grid,
        in_specs=[
            pl.BlockSpec((QB, D), lambda j, i: (i, 0)),
            pl.BlockSpec((KB, D), lambda j, i: (j, 0)),
        ],
        out_specs=pl.BlockSpec((QB, KB), lambda j, i: (i, j)),
        out_shape=jax.ShapeDtypeStruct((Q, N), jnp.float32),
    )(queries, keys)


def _extract_top30(candbuf, w, iota16):
    nv = (w + 15) // 16
    bigv = jnp.full((16,), BIG)

    def scan_min(i, acc):
        v = candbuf[pl.ds(i * 16, 16)]
        lm = (i * 16 + iota16) < w
        return jnp.minimum(acc, jnp.where(lm, v, bigv))

    def cond(state):
        remaining, _, _, _ = state
        return remaining > 0

    def body(state):
        remaining, o0, o1, _ = state
        m = jnp.min(lax.fori_loop(0, nv, scan_min, bigv))
        mv = jnp.full((16,), m)

        def rm_body(i, c):
            v = candbuf[pl.ds(i * 16, 16)]
            lm = (i * 16 + iota16) < w
            vm = jnp.where(lm, v, bigv)
            eq = vm == mv
            candbuf[pl.ds(i * 16, 16)] = jnp.where(eq, bigv, vm)
            return c + jnp.sum(eq.astype(jnp.int32))

        c = lax.fori_loop(0, nv, rm_body, jnp.int32(0))
        cc = jnp.minimum(c, remaining)
        pos0 = KNN - remaining
        sel0 = (iota16 >= pos0) & (iota16 < pos0 + cc)
        sel1 = (iota16 + 16 >= pos0) & (iota16 + 16 < pos0 + cc)
        o0 = jnp.where(sel0, mv, o0)
        o1 = jnp.where(sel1, mv, o1)
        return remaining - cc, o0, o1, m

    zero = jnp.zeros((16,), jnp.float32)
    state = (jnp.int32(KNN), zero, zero, jnp.float32(0.0))
    _, o0, o1, last_m = lax.while_loop(cond, body, state)
    return o0, o1, last_m


HALF = N // 2
IDXM = 16 * CAPL - 1


def _sc_body(sq_hbm, out_hbm, rowbuf, candbuf, buf2, outbuf, sem):
    wid = lax.axis_index("s") * 2 + lax.axis_index("c")
    iota16 = lax.iota(jnp.int32, 16)
    bigv = jnp.full((16,), BIG)
    zi = jnp.zeros((16,), jnp.int32)
    row0 = wid * ROWS_PER_W

    def row_body(j, carry):
        pltpu.async_copy(sq_hbm.at[row0 + j], rowbuf, sem).wait()

        @plsc.parallel_loop(0, HALF // 32, unroll=8, carry=(bigv, bigv))
        def p1_accs(i, accs):
            a0, a1 = accs
            a0 = jnp.minimum(a0, rowbuf[pl.ds(i * 16, 16)])
            a1 = jnp.minimum(a1, rowbuf[pl.ds(HALF // 2 + i * 16, 16)])
            return a0, a1

        a0, a1 = p1_accs
        m32 = jnp.maximum(jnp.max(a0), jnp.max(a1))
        mv = jnp.full((16,), m32)

        @plsc.parallel_loop(0, N // 16, unroll=8, carry=zi)
        def p2_cnt(i, cnt):
            v = rowbuf[pl.ds(i * 16, 16)]
            mask = v <= mv
            idx = (iota16 + (cnt << 4)) & IDXM
            plsc.store_scatter(candbuf, [idx], v, mask=mask)
            return cnt + mask.astype(jnp.int32)

        cnt = p2_cnt
        maxc = jnp.max(cnt)

        def fast_path():
            def cb(jv, w):
                v = candbuf[pl.ds(jv * 16, 16)]
                valid = cnt > jv
                pos = jnp.cumsum(valid.astype(jnp.int32))
                plsc.store_scatter(buf2, [(w - 1) + pos], v, mask=valid)
                return w + jnp.max(pos)

            w = lax.fori_loop(0, maxc, cb, jnp.int32(0))
            return _extract_top30(buf2, w, iota16)

        def slow_path():
            def append4(base, w):
                for t in range(4):
                    v = rowbuf[pl.ds(base + t * 16, 16)]
                    mask = v <= mv
                    pos = jnp.cumsum(mask.astype(jnp.int32))
                    plsc.store_scatter(buf2, [(w - 1) + pos], v, mask=mask)
                    w = w + jnp.max(pos)
                return w

            def compress(w):
                o0, o1, _ = _extract_top30(buf2, w, iota16)
                buf2[pl.ds(0, 16)] = o0
                buf2[pl.ds(16, 16)] = o1
                return jnp.int32(KNN)

            def p2e(g, w):
                base = g * 64
                v0 = rowbuf[pl.ds(base, 16)]
                v1 = rowbuf[pl.ds(base + 16, 16)]
                v2 = rowbuf[pl.ds(base + 32, 16)]
                v3 = rowbuf[pl.ds(base + 48, 16)]
                mn = jnp.minimum(jnp.minimum(v0, v1),
                                 jnp.minimum(v2, v3))
                hit = jnp.any(mn <= mv)
                w = lax.cond(hit, lambda w_: append4(base, w_),
                             lambda w_: w_, w)
                w = lax.cond(w >= CAP, compress, lambda w_: w_, w)
                return w

            w = lax.fori_loop(0, N // 64, p2e, jnp.int32(0))
            return _extract_top30(buf2, w, iota16)

        o0, o1, _ = lax.cond(maxc <= CAPL, fast_path, slow_path)
        outbuf[pl.ds(0, 16)] = o0
        outbuf[pl.ds(16, 16)] = o1
        pltpu.sync_copy(outbuf, out_hbm.at[row0 + j])
        return carry

    lax.fori_loop(0, ROWS_PER_W, row_body, jnp.int32(0))


def _sc_top30(sq):
    mesh = plsc.VectorSubcoreMesh(core_axis_name="c", subcore_axis_name="s")
    f = functools.partial(
        pl.kernel, _sc_body, mesh=mesh,
        compiler_params=pltpu.CompilerParams(needs_layout_passes=False),
        out_type=jax.ShapeDtypeStruct((Q, 32), jnp.float32),
        scratch_types=[
            pltpu.VMEM((N,), jnp.float32),
            pltpu.VMEM((16 * CAPL,), jnp.float32),
            pltpu.VMEM((CBUF,), jnp.float32),
            pltpu.VMEM((32,), jnp.float32),
            pltpu.SemaphoreType.DMA,
        ],
    )
    return f()(sq)


def _final_body(t30_ref, q_ref, pW1_ref, pb1_ref, pW2_ref, pb2_ref, pW3_ref,
                pb3_ref, tW1_ref, tb1_ref, tW2_ref, tb2_ref, tW3_ref, tb3_ref,
                out_ref):
    eps = 0.001
    L = 5.0
    dm0 = 0.001
    t30 = t30_ref[...]
    kmask = lax.broadcasted_iota(jnp.int32, (Q, 32), 1) < KNN
    d = jnp.sqrt(jnp.maximum(t30, 1e-12))
    d2 = d * d
    dm = 0.99 * dm0 + 0.01 * (
        jnp.sum(jnp.where(kmask, d2, 0.0), axis=1, keepdims=True) / KNN)
    ksum = jnp.sum(jnp.where(kmask, eps / (d2 / dm + eps), 0.0),
                   axis=1, keepdims=True)
    episodic = 1.0 / jnp.sqrt(ksum + eps)

    q = q_ref[...]

    def mlp(W1, b1, W2, b2, W3, b3s):
        h = lax.dot_general(q, W1, (((1,), (1,)), ((), ())),
                            preferred_element_type=jnp.float32) + b1
        h = jnp.maximum(h, 0.0)
        h = lax.dot_general(h, W2, (((1,), (1,)), ((), ())),
                            preferred_element_type=jnp.float32) + b2
        h = jnp.maximum(h, 0.0)
        return jnp.sum(h * W3, axis=1, keepdims=True) + b3s

    t_out = mlp(tW1_ref[...], tb1_ref[...], tW2_ref[...], tb2_ref[...],
                tW3_ref[...], tb3_ref[0, 0])
    p_out = mlp(pW1_ref[...], pb1_ref[...], pW2_ref[...], pb2_ref[...],
                pW3_ref[...], pb3_ref[0, 0])
    pe = (t_out - p_out) ** 2
    mu = jnp.mean(pe)
    var = jnp.mean((pe - mu) ** 2)
    std = jnp.sqrt(var + 1e-8)
    alpha = jnp.clip(pe / std, 1.0, L)
    out_ref[...] = episodic * alpha


def _final(t30, queries, pW1, pb1, pW2, pb2, pW3, pb3,
           tW1, tb1, tW2, tb2, tW3, tb3):
    args = (t30, queries,
            pW1, pb1.reshape(1, -1), pW2, pb2.reshape(1, -1),
            pW3, pb3.reshape(1, -1),
            tW1, tb1.reshape(1, -1), tW2, tb2.reshape(1, -1),
            tW3, tb3.reshape(1, -1))
    out = pl.pallas_call(
        _final_body,
        out_shape=jax.ShapeDtypeStruct((Q, 1), jnp.float32),
    )(*args)
    return out.reshape(Q)


def kernel(queries, keys, pW1, pb1, pW2, pb2, pW3, pb3,
           tW1, tb1, tW2, tb2, tW3, tb3):
    sq = _sq_distances(queries, keys)
    t30 = _sc_top30(sq)
    return _final(t30, queries, pW1, pb1, pW2, pb2, pW3, pb3,
                  tW1, tb1, tW2, tb2, tW3, tb3)


# speedup vs baseline: 1.7307x; 1.0801x over previous
import functools

import jax
import jax.numpy as jnp
from jax import lax
from jax.experimental import pallas as pl
from jax.experimental.pallas import tpu as pltpu
from jax.experimental.pallas import tpu_sc as plsc

Q = 1024
N = 65536
D = 256
QB = 256
KB = 2048

KNN = 30
NW = 32
ROWS_PER_W = Q // NW
CAPL = 64
CAP = 224
CBUF = 16 * CAPL + 16
BIG = 3.0e38


def _dist_body(q_ref, k_ref, out_ref):
    q = q_ref[...]
    k = k_ref[...]
    q2 = jnp.sum(q * q, axis=1, keepdims=True)
    ones = jnp.ones((8, D), jnp.float32)
    k2row = lax.dot_general(ones, k * k, (((1,), (1,)), ((), ())),
                            preferred_element_type=jnp.float32)
    qk = lax.dot_general(q, k, (((1,), (1,)), ((), ())),
                         preferred_element_type=jnp.float32)
    out_ref[...] = (q2 + k2row[0:1, :]) - 2.0 * qk


def _sq_distances(queries, keys):
    grid = (N // KB, Q // QB)
    return pl.pallas_call(
        _dist_body,
        grid=grid,
        in_specs=[
            pl.BlockSpec((QB, D), lambda j, i: (i, 0)),
            pl.BlockSpec((KB, D), lambda j, i: (j, 0)),
        ],
        out_specs=pl.BlockSpec((QB, KB), lambda j, i: (i, j)),
        out_shape=jax.ShapeDtypeStruct((Q, N), jnp.float32),
    )(queries, keys)


def _extract_top30(candbuf, w, iota16):
    nv = (w + 15) // 16
    bigv = jnp.full((16,), BIG)

    def scan_min(i, acc):
        v = candbuf[pl.ds(i * 16, 16)]
        lm = (i * 16 + iota16) < w
        return jnp.minimum(acc, jnp.where(lm, v, bigv))

    def cond(state):
        remaining, _, _, _ = state
        return remaining > 0

    def body(state):
        remaining, o0, o1, _ = state
        m = jnp.min(lax.fori_loop(0, nv, scan_min, bigv))
        mv = jnp.full((16,), m)

        def rm_body(i, c):
            v = candbuf[pl.ds(i * 16, 16)]
            lm = (i * 16 + iota16) < w
            vm = jnp.where(lm, v, bigv)
            eq = vm == mv
            candbuf[pl.ds(i * 16, 16)] = jnp.where(eq, bigv, vm)
            return c + jnp.sum(eq.astype(jnp.int32))

        c = lax.fori_loop(0, nv, rm_body, jnp.int32(0))
        cc = jnp.minimum(c, remaining)
        pos0 = KNN - remaining
        sel0 = (iota16 >= pos0) & (iota16 < pos0 + cc)
        sel1 = (iota16 + 16 >= pos0) & (iota16 + 16 < pos0 + cc)
        o0 = jnp.where(sel0, mv, o0)
        o1 = jnp.where(sel1, mv, o1)
        return remaining - cc, o0, o1, m

    zero = jnp.zeros((16,), jnp.float32)
    state = (jnp.int32(KNN), zero, zero, jnp.float32(0.0))
    _, o0, o1, last_m = lax.while_loop(cond, body, state)
    return o0, o1, last_m


HALF = N // 2
IDXM = 16 * CAPL - 1


def _sc_body(sq_hbm, out_hbm, rowbuf, candbuf, buf2, outbuf, sem):
    wid = lax.axis_index("s") * 2 + lax.axis_index("c")
    iota16 = lax.iota(jnp.int32, 16)
    bigv = jnp.full((16,), BIG)
    zi = jnp.zeros((16,), jnp.int32)
    row0 = wid * ROWS_PER_W

    def row_body(j, carry):
        pltpu.async_copy(sq_hbm.at[row0 + j], rowbuf, sem).wait()

        @plsc.parallel_loop(0, N // 32, unroll=8, carry=(bigv, bigv))
        def p1_accs(i, accs):
            a0, a1 = accs
            a0 = jnp.minimum(a0, rowbuf[pl.ds(i * 16, 16)])
            a1 = jnp.minimum(a1, rowbuf[pl.ds(HALF + i * 16, 16)])
            return a0, a1

        a0, a1 = p1_accs
        m32 = jnp.maximum(jnp.max(a0), jnp.max(a1))
        mv = jnp.full((16,), m32)

        @plsc.parallel_loop(0, N // 16, unroll=8, carry=zi)
        def p2_cnt(i, cnt):
            v = rowbuf[pl.ds(i * 16, 16)]
            mask = v <= mv
            idx = (iota16 + (cnt << 4)) & IDXM
            plsc.store_scatter(candbuf, [idx], v, mask=mask)
            return cnt + mask.astype(jnp.int32)

        cnt = p2_cnt
        maxc = jnp.max(cnt)

        def fast_path():
            def cb(jv, w):
                v = candbuf[pl.ds(jv * 16, 16)]
                valid = cnt > jv
                pos = jnp.cumsum(valid.astype(jnp.int32))
                plsc.store_scatter(buf2, [(w - 1) + pos], v, mask=valid)
                return w + jnp.max(pos)

            w = lax.fori_loop(0, maxc, cb, jnp.int32(0))
            return _extract_top30(buf2, w, iota16)

        def slow_path():
            def append4(base, w):
                for t in range(4):
                    v = rowbuf[pl.ds(base + t * 16, 16)]
                    mask = v <= mv
                    pos = jnp.cumsum(mask.astype(jnp.int32))
                    plsc.store_scatter(buf2, [(w - 1) + pos], v, mask=mask)
                    w = w + jnp.max(pos)
                return w

            def compress(w):
                o0, o1, _ = _extract_top30(buf2, w, iota16)
                buf2[pl.ds(0, 16)] = o0
                buf2[pl.ds(16, 16)] = o1
                return jnp.int32(KNN)

            def p2e(g, w):
                base = g * 64
                v0 = rowbuf[pl.ds(base, 16)]
                v1 = rowbuf[pl.ds(base + 16, 16)]
                v2 = rowbuf[pl.ds(base + 32, 16)]
                v3 = rowbuf[pl.ds(base + 48, 16)]
                mn = jnp.minimum(jnp.minimum(v0, v1),
                                 jnp.minimum(v2, v3))
                hit = jnp.any(mn <= mv)
                w = lax.cond(hit, lambda w_: append4(base, w_),
                             lambda w_: w_, w)
                w = lax.cond(w >= CAP, compress, lambda w_: w_, w)
                return w

            w = lax.fori_loop(0, N // 64, p2e, jnp.int32(0))
            return _extract_top30(buf2, w, iota16)

        o0, o1, _ = lax.cond(maxc <= CAPL, fast_path, slow_path)
        outbuf[pl.ds(0, 16)] = o0
        outbuf[pl.ds(16, 16)] = o1
        pltpu.sync_copy(outbuf, out_hbm.at[row0 + j])
        return carry

    lax.fori_loop(0, ROWS_PER_W, row_body, jnp.int32(0))


def _sc_top30(sq):
    mesh = plsc.VectorSubcoreMesh(core_axis_name="c", subcore_axis_name="s")
    f = functools.partial(
        pl.kernel, _sc_body, mesh=mesh,
        compiler_params=pltpu.CompilerParams(needs_layout_passes=False),
        out_type=jax.ShapeDtypeStruct((Q, 32), jnp.float32),
        scratch_types=[
            pltpu.VMEM((N,), jnp.float32),
            pltpu.VMEM((16 * CAPL,), jnp.float32),
            pltpu.VMEM((CBUF,), jnp.float32),
            pltpu.VMEM((32,), jnp.float32),
            pltpu.SemaphoreType.DMA,
        ],
    )
    return f()(sq)


def _final_body(t30_ref, q_ref, pW1_ref, pb1_ref, pW2_ref, pb2_ref, pW3_ref,
                pb3_ref, tW1_ref, tb1_ref, tW2_ref, tb2_ref, tW3_ref, tb3_ref,
                out_ref):
    eps = 0.001
    L = 5.0
    dm0 = 0.001
    t30 = t30_ref[...]
    kmask = lax.broadcasted_iota(jnp.int32, (Q, 32), 1) < KNN
    d = jnp.sqrt(jnp.maximum(t30, 1e-12))
    d2 = d * d
    dm = 0.99 * dm0 + 0.01 * (
        jnp.sum(jnp.where(kmask, d2, 0.0), axis=1, keepdims=True) / KNN)
    ksum = jnp.sum(jnp.where(kmask, eps / (d2 / dm + eps), 0.0),
                   axis=1, keepdims=True)
    episodic = 1.0 / jnp.sqrt(ksum + eps)

    q = q_ref[...]

    def mlp(W1, b1, W2, b2, W3, b3s):
        h = lax.dot_general(q, W1, (((1,), (1,)), ((), ())),
                            preferred_element_type=jnp.float32) + b1
        h = jnp.maximum(h, 0.0)
        h = lax.dot_general(h, W2, (((1,), (1,)), ((), ())),
                            preferred_element_type=jnp.float32) + b2
        h = jnp.maximum(h, 0.0)
        return jnp.sum(h * W3, axis=1, keepdims=True) + b3s

    t_out = mlp(tW1_ref[...], tb1_ref[...], tW2_ref[...], tb2_ref[...],
                tW3_ref[...], tb3_ref[0, 0])
    p_out = mlp(pW1_ref[...], pb1_ref[...], pW2_ref[...], pb2_ref[...],
                pW3_ref[...], pb3_ref[0, 0])
    pe = (t_out - p_out) ** 2
    mu = jnp.mean(pe)
    var = jnp.mean((pe - mu) ** 2)
    std = jnp.sqrt(var + 1e-8)
    alpha = jnp.clip(pe / std, 1.0, L)
    out_ref[...] = episodic * alpha


def _final(t30, queries, pW1, pb1, pW2, pb2, pW3, pb3,
           tW1, tb1, tW2, tb2, tW3, tb3):
    args = (t30, queries,
            pW1, pb1.reshape(1, -1), pW2, pb2.reshape(1, -1),
            pW3, pb3.reshape(1, -1),
            tW1, tb1.reshape(1, -1), tW2, tb2.reshape(1, -1),
            tW3, tb3.reshape(1, -1))
    out = pl.pallas_call(
        _final_body,
        out_shape=jax.ShapeDtypeStruct((Q, 1), jnp.float32),
    )(*args)
    return out.reshape(Q)


def kernel(queries, keys, pW1, pb1, pW2, pb2, pW3, pb3,
           tW1, tb1, tW2, tb2, tW3, tb3):
    sq = _sq_distances(queries, keys)
    t30 = _sc_top30(sq)
    return _final(t30, queries, pW1, pb1, pW2, pb2, pW3, pb3,
                  tW1, tb1, tW2, tb2, tW3, tb3)


# speedup vs baseline: 1.7324x; 1.0010x over previous
import functools

import jax
import jax.numpy as jnp
from jax import lax
from jax.experimental import pallas as pl
from jax.experimental.pallas import tpu as pltpu
from jax.experimental.pallas import tpu_sc as plsc

Q = 1024
N = 65536
D = 256
QB = 256
KB = 2048

KNN = 30
NW = 32
ROWS_PER_W = Q // NW
CAPL = 64
CAP = 224
CBUF = 16 * CAPL + 16
BIG = 3.0e38


def _dist_body(q_ref, k_ref, out_ref):
    q = q_ref[...]
    k = k_ref[...]
    q2 = jnp.sum(q * q, axis=1, keepdims=True)
    ones = jnp.ones((8, D), jnp.float32)
    k2row = lax.dot_general(ones, k * k, (((1,), (1,)), ((), ())),
                            preferred_element_type=jnp.float32)
    qk = lax.dot_general(q, k, (((1,), (1,)), ((), ())),
                         preferred_element_type=jnp.float32)
    out_ref[...] = (q2 + k2row[0:1, :]) - 2.0 * qk


def _sq_distances(queries, keys):
    grid = (N // KB, Q // QB)
    return pl.pallas_call(
        _dist_body,
        grid=grid,
        in_specs=[
            pl.BlockSpec((QB, D), lambda j, i: (i, 0)),
            pl.BlockSpec((KB, D), lambda j, i: (j, 0)),
        ],
        out_specs=pl.BlockSpec((QB, KB), lambda j, i: (i, j)),
        out_shape=jax.ShapeDtypeStruct((Q, N), jnp.float32),
    )(queries, keys)


def _extract_top30(candbuf, w, iota16):
    nv = (w + 15) // 16
    bigv = jnp.full((16,), BIG)

    def scan_min(i, acc):
        v = candbuf[pl.ds(i * 16, 16)]
        lm = (i * 16 + iota16) < w
        return jnp.minimum(acc, jnp.where(lm, v, bigv))

    def cond(state):
        remaining, _, _, _ = state
        return remaining > 0

    def body(state):
        remaining, o0, o1, _ = state
        m = jnp.min(lax.fori_loop(0, nv, scan_min, bigv))
        mv = jnp.full((16,), m)

        def rm_body(i, c):
            v = candbuf[pl.ds(i * 16, 16)]
            lm = (i * 16 + iota16) < w
            vm = jnp.where(lm, v, bigv)
            eq = vm == mv
            candbuf[pl.ds(i * 16, 16)] = jnp.where(eq, bigv, vm)
            return c + jnp.sum(eq.astype(jnp.int32))

        c = lax.fori_loop(0, nv, rm_body, jnp.int32(0))
        cc = jnp.minimum(c, remaining)
        pos0 = KNN - remaining
        sel0 = (iota16 >= pos0) & (iota16 < pos0 + cc)
        sel1 = (iota16 + 16 >= pos0) & (iota16 + 16 < pos0 + cc)
        o0 = jnp.where(sel0, mv, o0)
        o1 = jnp.where(sel1, mv, o1)
        return remaining - cc, o0, o1, m

    zero = jnp.zeros((16,), jnp.float32)
    state = (jnp.int32(KNN), zero, zero, jnp.float32(0.0))
    _, o0, o1, last_m = lax.while_loop(cond, body, state)
    return o0, o1, last_m


HALF = N // 2
IDXM = 16 * CAPL - 1


def _sc_body(sq_hbm, out_hbm, rowbuf, candbuf, buf2, outbuf, sem):
    wid = lax.axis_index("s") * 2 + lax.axis_index("c")
    iota16 = lax.iota(jnp.int32, 16)
    bigv = jnp.full((16,), BIG)
    zi = jnp.zeros((16,), jnp.int32)
    row0 = wid * ROWS_PER_W

    def row_body(j, carry):
        pltpu.async_copy(sq_hbm.at[row0 + j], rowbuf, sem).wait()

        @plsc.parallel_loop(0, N // 32, unroll=8, carry=(bigv, bigv))
        def p1_accs(i, accs):
            a0, a1 = accs
            a0 = jnp.minimum(a0, rowbuf[pl.ds(i * 16, 16)])
            a1 = jnp.minimum(a1, rowbuf[pl.ds(HALF + i * 16, 16)])
            return a0, a1

        a0, a1 = p1_accs
        m32 = jnp.maximum(jnp.max(a0), jnp.max(a1))
        mv = jnp.full((16,), m32)

        @plsc.parallel_loop(0, N // 16, unroll=8, carry=zi)
        def p2_cnt(i, cnt):
            v = rowbuf[pl.ds(i * 16, 16)]
            mask = v <= mv
            idx = (iota16 + (cnt << 4)) & IDXM
            plsc.store_scatter(candbuf, [idx], v, mask=mask)
            return cnt + mask.astype(jnp.int32)

        cnt = p2_cnt
        maxc = jnp.max(cnt)

        def fast_path():
            def cb(jv, w):
                v = candbuf[pl.ds(jv * 16, 16)]
                valid = cnt > jv
                pos = jnp.cumsum(valid.astype(jnp.int32))
                plsc.store_scatter(buf2, [(w - 1) + pos], v, mask=valid)
                return w + jnp.max(pos)

            w = lax.fori_loop(0, maxc, cb, jnp.int32(0))
            return _extract_top30(buf2, w, iota16)

        def slow_path():
            def append4(base, w):
                for t in range(4):
                    v = rowbuf[pl.ds(base + t * 16, 16)]
                    mask = v <= mv
                    pos = jnp.cumsum(mask.astype(jnp.int32))
                    plsc.store_scatter(buf2, [(w - 1) + pos], v, mask=mask)
                    w = w + jnp.max(pos)
                return w

            def compress(w):
                o0, o1, _ = _extract_top30(buf2, w, iota16)
                buf2[pl.ds(0, 16)] = o0
                buf2[pl.ds(16, 16)] = o1
                return jnp.int32(KNN)

            def p2e(g, w):
                base = g * 64
                v0 = rowbuf[pl.ds(base, 16)]
                v1 = rowbuf[pl.ds(base + 16, 16)]
                v2 = rowbuf[pl.ds(base + 32, 16)]
                v3 = rowbuf[pl.ds(base + 48, 16)]
                mn = jnp.minimum(jnp.minimum(v0, v1),
                                 jnp.minimum(v2, v3))
                hit = jnp.any(mn <= mv)
                w = lax.cond(hit, lambda w_: append4(base, w_),
                             lambda w_: w_, w)
                w = lax.cond(w >= CAP, compress, lambda w_: w_, w)
                return w

            w = lax.fori_loop(0, N // 64, p2e, jnp.int32(0))
            return _extract_top30(buf2, w, iota16)

        o0, o1, _ = lax.cond(maxc <= CAPL, fast_path, slow_path)
        outbuf[pl.ds(0, 16)] = o0
        outbuf[pl.ds(16, 16)] = o1
        pltpu.sync_copy(outbuf, out_hbm.at[row0 + j])
        return carry

    lax.fori_loop(0, ROWS_PER_W, row_body, jnp.int32(0))


def _sc_top30(sq):
    mesh = plsc.VectorSubcoreMesh(core_axis_name="c", subcore_axis_name="s")
    f = functools.partial(
        pl.kernel, _sc_body, mesh=mesh,
        compiler_params=pltpu.CompilerParams(needs_layout_passes=False),
        out_type=jax.ShapeDtypeStruct((Q, 32), jnp.float32),
        scratch_types=[
            pltpu.VMEM((N,), jnp.float32),
            pltpu.VMEM((16 * CAPL,), jnp.float32),
            pltpu.VMEM((CBUF,), jnp.float32),
            pltpu.VMEM((32,), jnp.float32),
            pltpu.SemaphoreType.DMA,
        ],
    )
    return f()(sq)


def _alpha_body(q_ref, pW1_ref, pb1_ref, pW2_ref, pb2_ref, pW3_ref,
                pb3_ref, tW1_ref, tb1_ref, tW2_ref, tb2_ref, tW3_ref,
                tb3_ref, out_ref):
    L = 5.0
    q = q_ref[...]

    def mlp(W1, b1, W2, b2, W3, b3s):
        h = lax.dot_general(q, W1, (((1,), (1,)), ((), ())),
                            preferred_element_type=jnp.float32) + b1
        h = jnp.maximum(h, 0.0)
        h = lax.dot_general(h, W2, (((1,), (1,)), ((), ())),
                            preferred_element_type=jnp.float32) + b2
        h = jnp.maximum(h, 0.0)
        return jnp.sum(h * W3, axis=1, keepdims=True) + b3s

    t_out = mlp(tW1_ref[...], tb1_ref[...], tW2_ref[...], tb2_ref[...],
                tW3_ref[...], tb3_ref[0, 0])
    p_out = mlp(pW1_ref[...], pb1_ref[...], pW2_ref[...], pb2_ref[...],
                pW3_ref[...], pb3_ref[0, 0])
    pe = (t_out - p_out) ** 2
    mu = jnp.mean(pe)
    var = jnp.mean((pe - mu) ** 2)
    std = jnp.sqrt(var + 1e-8)
    out_ref[...] = jnp.clip(pe / std, 1.0, L)


def _alpha(queries, pW1, pb1, pW2, pb2, pW3, pb3,
           tW1, tb1, tW2, tb2, tW3, tb3):
    args = (queries,
            pW1, pb1.reshape(1, -1), pW2, pb2.reshape(1, -1),
            pW3, pb3.reshape(1, -1),
            tW1, tb1.reshape(1, -1), tW2, tb2.reshape(1, -1),
            tW3, tb3.reshape(1, -1))
    return pl.pallas_call(
        _alpha_body,
        out_shape=jax.ShapeDtypeStruct((Q, 1), jnp.float32),
    )(*args)


def _combine_body(t30_ref, alpha_ref, out_ref):
    eps = 0.001
    dm0 = 0.001
    t30 = t30_ref[...]
    kmask = lax.broadcasted_iota(jnp.int32, (Q, 32), 1) < KNN
    d = jnp.sqrt(jnp.maximum(t30, 1e-12))
    d2 = d * d
    dm = 0.99 * dm0 + 0.01 * (
        jnp.sum(jnp.where(kmask, d2, 0.0), axis=1, keepdims=True) / KNN)
    ksum = jnp.sum(jnp.where(kmask, eps / (d2 / dm + eps), 0.0),
                   axis=1, keepdims=True)
    episodic = 1.0 / jnp.sqrt(ksum + eps)
    out_ref[...] = episodic * alpha_ref[...]


def kernel(queries, keys, pW1, pb1, pW2, pb2, pW3, pb3,
           tW1, tb1, tW2, tb2, tW3, tb3):
    alpha = _alpha(queries, pW1, pb1, pW2, pb2, pW3, pb3,
                   tW1, tb1, tW2, tb2, tW3, tb3)
    sq = _sq_distances(queries, keys)
    t30 = _sc_top30(sq)
    out = pl.pallas_call(
        _combine_body,
        out_shape=jax.ShapeDtypeStruct((Q, 1), jnp.float32),
    )(t30, alpha)
    return out.reshape(Q)
